# baseline (speedup 1.0000x reference)
import jax
import jax.numpy as jnp
from jax.experimental import pallas as pl
from jax.experimental.pallas import tpu as pltpu

N = 10000
NC0 = 4992
NC1 = N - NC0
NFEAT = 128
NHID = 16
NCLS = 2
RA = 400
NBLKA = N // RA
RB = 1000
NBLKB = N // RB


def _body_a(adj_ref, x_ref, Wc_ref, bc_ref, adjL_ref, adjR_ref, qs_ref,
            packed_ref):
    adj = adj_ref[...]
    a16 = adj.astype(jnp.bfloat16)
    adjL_ref[...] = a16[:, :NC0]
    adjR_ref[...] = a16[:, NC0:]
    deg = jnp.sum(adj, axis=1, keepdims=True)
    dinv = jnp.where(deg > 0.0,
                     jax.lax.rsqrt(jnp.maximum(deg, 1e-12)), 0.0)
    xb = x_ref[...]
    W0 = Wc_ref[0]
    W1 = Wc_ref[1]
    W2c = Wc_ref[2]
    P = jnp.dot(xb, W1, preferred_element_type=jnp.float32)
    Q = jnp.dot(xb, W2c, preferred_element_type=jnp.float32)
    base = jnp.dot(xb, W0 - W2c, preferred_element_type=jnp.float32)
    qs_ref[...] = (dinv * Q).astype(jnp.bfloat16)
    packed_ref[...] = jnp.concatenate(
        [dinv * P, base + bc_ref[...], jnp.broadcast_to(dinv, (RA, NHID))],
        axis=1)


def _body_b(adjL_ref, adjR_ref, qs_ref, packed_ref, W2p_ref, b2p_ref,
            out_ref, sc_ref, sup_ref, macc_ref):
    p = pl.program_id(0)
    i = pl.program_id(1)
    sl = pl.ds(i * RB, RB)

    @pl.when(p == 0)
    def _cheb():
        U = (jnp.dot(adjL_ref[...], qs_ref[:NC0, :],
                     preferred_element_type=jnp.float32)
             + jnp.dot(adjR_ref[...], qs_ref[NC0:, :],
                       preferred_element_type=jnp.float32))
        pd = packed_ref[sl, 0:NHID]
        dinv = packed_ref[sl, 2 * NHID:3 * NHID]
        sc_ref[sl, :] = 2.0 * (dinv * dinv) * U - pd

    @pl.when(p == 1)
    def _hidden():
        T = (jnp.dot(adjL_ref[...], sc_ref[:NC0, :].astype(jnp.bfloat16),
                     preferred_element_type=jnp.float32)
             + jnp.dot(adjR_ref[...], sc_ref[NC0:, :].astype(jnp.bfloat16),
                       preferred_element_type=jnp.float32))
        base = packed_ref[sl, NHID:2 * NHID]
        dinv = packed_ref[sl, 2 * NHID:3 * NHID]
        h = jnp.maximum(base + dinv * T, 0.0)
        sup_ref[sl, :] = jnp.dot(h, W2p_ref[...],
                                 preferred_element_type=jnp.float32)

    @pl.when(p == 2)
    def _pool():
        O = (jnp.dot(adjL_ref[...], sup_ref[:NC0, :].astype(jnp.bfloat16),
                     preferred_element_type=jnp.float32)
             + jnp.dot(adjR_ref[...], sup_ref[NC0:, :].astype(jnp.bfloat16),
                       preferred_element_type=jnp.float32))
        m = jnp.max(O, axis=0, keepdims=True)

        @pl.when(i == 0)
        def _():
            macc_ref[...] = m

        @pl.when(i > 0)
        def _():
            macc_ref[...] = jnp.maximum(macc_ref[...], m)

        @pl.when(i == NBLKB - 1)
        def _():
            out_ref[...] = macc_ref[...] + b2p_ref[...]


def kernel(x, adj, W_cheb, b_cheb, W2, b2):
    bc2 = b_cheb.reshape(1, NHID)
    W2p = jnp.zeros((NHID, NHID), jnp.float32).at[:, :NCLS].set(W2)
    b2p = jnp.zeros((1, NHID), jnp.float32).at[0, :NCLS].set(b2)

    adjL, adjR, qs, packed = pl.pallas_call(
        _body_a,
        grid=(NBLKA,),
        in_specs=[
            pl.BlockSpec((RA, N), lambda i: (i, 0)),
            pl.BlockSpec((RA, NFEAT), lambda i: (i, 0)),
            pl.BlockSpec((3, NFEAT, NHID), lambda i: (0, 0, 0)),
            pl.BlockSpec((1, NHID), lambda i: (0, 0)),
        ],
        out_specs=[
            pl.BlockSpec((RA, NC0), lambda i: (i, 0)),
            pl.BlockSpec((RA, NC1), lambda i: (i, 0)),
            pl.BlockSpec((RA, NHID), lambda i: (i, 0)),
            pl.BlockSpec((RA, 3 * NHID), lambda i: (i, 0)),
        ],
        out_shape=[
            jax.ShapeDtypeStruct((N, NC0), jnp.bfloat16),
            jax.ShapeDtypeStruct((N, NC1), jnp.bfloat16),
            jax.ShapeDtypeStruct((N, NHID), jnp.bfloat16),
            jax.ShapeDtypeStruct((N, 3 * NHID), jnp.float32),
        ],
        compiler_params=pltpu.CompilerParams(
            dimension_semantics=("arbitrary",),
        ),
    )(adj, x, W_cheb, bc2)

    out = pl.pallas_call(
        _body_b,
        grid=(3, NBLKB),
        in_specs=[
            pl.BlockSpec((RB, NC0), lambda p, i: (i, 0)),
            pl.BlockSpec((RB, NC1), lambda p, i: (i, 0)),
            pl.BlockSpec((N, NHID), lambda p, i: (0, 0)),
            pl.BlockSpec((N, 3 * NHID), lambda p, i: (0, 0)),
            pl.BlockSpec((NHID, NHID), lambda p, i: (0, 0)),
            pl.BlockSpec((1, NHID), lambda p, i: (0, 0)),
        ],
        out_specs=pl.BlockSpec((1, NHID), lambda p, i: (0, 0)),
        out_shape=jax.ShapeDtypeStruct((1, NHID), jnp.float32),
        scratch_shapes=[
            pltpu.VMEM((N, NHID), jnp.float32),
            pltpu.VMEM((N, NHID), jnp.float32),
            pltpu.VMEM((1, NHID), jnp.float32),
        ],
        compiler_params=pltpu.CompilerParams(
            dimension_semantics=("arbitrary", "arbitrary"),
        ),
    )(adjL, adjR, qs, packed, W2p, b2p)
    return out[:, :NCLS].reshape(1, 1, NCLS)


# speedup vs baseline: 1.0031x; 1.0031x over previous
import jax
import jax.numpy as jnp
from jax.experimental import pallas as pl
from jax.experimental.pallas import tpu as pltpu

N = 10000
NFEAT = 128
NHID = 16
NCLS = 2
RA = 400
NBLKA = N // RA
RB = 1000
NBLKB = N // RB


def _body_a(adj_ref, x_ref, Wc_ref, bc_ref, adj16_ref, qs_ref, packed_ref):
    adj = adj_ref[...]
    adj16_ref[...] = adj.astype(jnp.bfloat16)
    deg = jnp.sum(adj, axis=1, keepdims=True)
    dinv = jnp.where(deg > 0.0,
                     jax.lax.rsqrt(jnp.maximum(deg, 1e-12)), 0.0)
    xb = x_ref[...]
    W0 = Wc_ref[0]
    W1 = Wc_ref[1]
    W2c = Wc_ref[2]
    P = jnp.dot(xb, W1, preferred_element_type=jnp.float32)
    Q = jnp.dot(xb, W2c, preferred_element_type=jnp.float32)
    base = jnp.dot(xb, W0 - W2c, preferred_element_type=jnp.float32)
    qs_ref[...] = (dinv * Q).astype(jnp.bfloat16)
    packed_ref[...] = jnp.concatenate(
        [dinv * P, base + bc_ref[...], jnp.broadcast_to(dinv, (RA, NHID))],
        axis=1)


def _body_b(adj16_ref, qs_ref, packed_ref, W2p_ref, b2p_ref,
            out_ref, sc_ref, sup_ref, macc_ref):
    p = pl.program_id(0)
    i = pl.program_id(1)
    sl = pl.ds(i * RB, RB)

    @pl.when(p == 0)
    def _cheb():
        U = jnp.dot(adj16_ref[...], qs_ref[...],
                    preferred_element_type=jnp.float32)
        pd = packed_ref[sl, 0:NHID]
        dinv = packed_ref[sl, 2 * NHID:3 * NHID]
        sc_ref[sl, :] = 2.0 * (dinv * dinv) * U - pd

    @pl.when(p == 1)
    def _hidden():
        T = jnp.dot(adj16_ref[...], sc_ref[...].astype(jnp.bfloat16),
                    preferred_element_type=jnp.float32)
        base = packed_ref[sl, NHID:2 * NHID]
        dinv = packed_ref[sl, 2 * NHID:3 * NHID]
        h = jnp.maximum(base + dinv * T, 0.0)
        sup_ref[sl, :] = jnp.dot(h, W2p_ref[...],
                                 preferred_element_type=jnp.float32)

    @pl.when(p == 2)
    def _pool():
        O = jnp.dot(adj16_ref[...], sup_ref[...].astype(jnp.bfloat16),
                    preferred_element_type=jnp.float32)
        m = jnp.max(O, axis=0, keepdims=True)

        @pl.when(i == 0)
        def _():
            macc_ref[...] = m

        @pl.when(i > 0)
        def _():
            macc_ref[...] = jnp.maximum(macc_ref[...], m)

        @pl.when(i == NBLKB - 1)
        def _():
            out_ref[...] = macc_ref[...] + b2p_ref[...]


def kernel(x, adj, W_cheb, b_cheb, W2, b2):
    bc2 = b_cheb.reshape(1, NHID)
    W2p = jnp.zeros((NHID, NHID), jnp.float32).at[:, :NCLS].set(W2)
    b2p = jnp.zeros((1, NHID), jnp.float32).at[0, :NCLS].set(b2)

    adj16, qs, packed = pl.pallas_call(
        _body_a,
        grid=(NBLKA,),
        in_specs=[
            pl.BlockSpec((RA, N), lambda i: (i, 0)),
            pl.BlockSpec((RA, NFEAT), lambda i: (i, 0)),
            pl.BlockSpec((3, NFEAT, NHID), lambda i: (0, 0, 0)),
            pl.BlockSpec((1, NHID), lambda i: (0, 0)),
        ],
        out_specs=[
            pl.BlockSpec((RA, N), lambda i: (i, 0)),
            pl.BlockSpec((RA, NHID), lambda i: (i, 0)),
            pl.BlockSpec((RA, 3 * NHID), lambda i: (i, 0)),
        ],
        out_shape=[
            jax.ShapeDtypeStruct((N, N), jnp.bfloat16),
            jax.ShapeDtypeStruct((N, NHID), jnp.bfloat16),
            jax.ShapeDtypeStruct((N, 3 * NHID), jnp.float32),
        ],
        compiler_params=pltpu.CompilerParams(
            dimension_semantics=("arbitrary",),
        ),
    )(adj, x, W_cheb, bc2)

    out = pl.pallas_call(
        _body_b,
        grid=(3, NBLKB),
        in_specs=[
            pl.BlockSpec((RB, N), lambda p, i: (i, 0)),
            pl.BlockSpec((N, NHID), lambda p, i: (0, 0)),
            pl.BlockSpec((N, 3 * NHID), lambda p, i: (0, 0)),
            pl.BlockSpec((NHID, NHID), lambda p, i: (0, 0)),
            pl.BlockSpec((1, NHID), lambda p, i: (0, 0)),
        ],
        out_specs=pl.BlockSpec((1, NHID), lambda p, i: (0, 0)),
        out_shape=jax.ShapeDtypeStruct((1, NHID), jnp.float32),
        scratch_shapes=[
            pltpu.VMEM((N, NHID), jnp.float32),
            pltpu.VMEM((N, NHID), jnp.float32),
            pltpu.VMEM((1, NHID), jnp.float32),
        ],
        compiler_params=pltpu.CompilerParams(
            dimension_semantics=("arbitrary", "arbitrary"),
        ),
    )(adj16, qs, packed, W2p, b2p)
    return out[:, :NCLS].reshape(1, 1, NCLS)
